# baseline (speedup 1.0000x reference)
import functools

import jax
import jax.numpy as jnp
from jax import lax
from jax.experimental import pallas as pl
from jax.experimental.pallas import tpu as pltpu
from jax.experimental.pallas import tpu_sc as plsc

_N = 10000
_E = 320000
_H = 128
_EPS = 1e-5

_NC = 2
_NS = 16
_NW = _NC * _NS
_CH = 128
_NCHUNK = 80
_NH = _NCHUNK // 2
_EP = _NW * _NCHUNK * _CH
_RPT = 632
_AGG_ROWS = _NS * _RPT


def _sc_agg_body(src_hbm, dst_hbm, table_hbm, zeros_hbm, out_hbm,
                 src_v, dst_v, rows_v, agg_sh, sem_a, sem_b):
    c = lax.axis_index("c")
    s = lax.axis_index("s")
    wid = c * _NS + s
    base = s * _RPT
    pltpu.sync_copy(zeros_hbm, agg_sh.at[pl.ds(base, _RPT)])
    plsc.subcore_barrier()

    for h in range(2):
        pltpu.sync_copy(src_hbm.at[wid, h], src_v)
        pltpu.sync_copy(dst_hbm.at[wid, h], dst_v)
        pltpu.async_copy(table_hbm.at[src_v.at[0]], rows_v.at[0], sem_a)

        def pairs(jp, carry):
            j0 = 2 * jp
            pltpu.make_async_copy(table_hbm.at[src_v.at[0]],
                                  rows_v.at[0], sem_a).wait()
            pltpu.async_copy(table_hbm.at[src_v.at[j0 + 1]],
                             rows_v.at[1], sem_b)
            pltpu.sync_copy(rows_v.at[0], agg_sh.at[dst_v.at[j0]], add=True)
            pltpu.make_async_copy(table_hbm.at[src_v.at[0]],
                                  rows_v.at[1], sem_b).wait()
            pltpu.async_copy(table_hbm.at[src_v.at[j0 + 2]],
                             rows_v.at[0], sem_a)
            pltpu.sync_copy(rows_v.at[1], agg_sh.at[dst_v.at[j0 + 1]],
                            add=True)
            return carry

        lax.fori_loop(0, _NH // 2, pairs, 0)
        pltpu.make_async_copy(table_hbm.at[src_v.at[0]], rows_v.at[0],
                              sem_a).wait()
    plsc.subcore_barrier()
    pltpu.sync_copy(agg_sh.at[pl.ds(base, _RPT)],
                    out_hbm.at[c, pl.ds(base, _RPT)])


_sc_agg = pl.kernel(
    _sc_agg_body,
    out_type=jax.ShapeDtypeStruct((_NC, _AGG_ROWS, _H), jnp.float32),
    mesh=plsc.VectorSubcoreMesh(core_axis_name="c", subcore_axis_name="s"),
    scratch_types=[
        pltpu.VMEM((_NH + 1, _CH), jnp.int32),
        pltpu.VMEM((_NH + 1, _CH), jnp.int32),
        pltpu.VMEM((2, _CH, _H), jnp.float32),
        pltpu.VMEM_SHARED((_AGG_ROWS, _H), jnp.float32),
        pltpu.SemaphoreType.DMA,
        pltpu.SemaphoreType.DMA,
    ],
)


def _style(h, wv, aW, ab, ns, noise):
    h = h + noise * ns
    st = jnp.dot(wv, aW, preferred_element_type=jnp.float32) + ab
    gamma = st[:, :_H]
    beta = st[:, _H:]
    mu = jnp.mean(h, axis=0, keepdims=True)
    d = h - mu
    var = jnp.mean(d * d, axis=0, keepdims=True)
    hn = d * lax.rsqrt(var + _EPS)
    o = gamma * hn + beta
    return jnp.where(o >= 0, o, 0.01 * o)


def _dense_body(parts_ref, x_ref, w_ref, edge_b_ref, c1W_ref, c1b_ref,
                c2W_ref, c2b_ref, nmW_ref, nmb_ref, nmaW_ref, nmab_ref,
                nmns_ref, nmnoise_ref, f1W_ref, f1b_ref, f1aW_ref, f1ab_ref,
                f1ns_ref, f1noise_ref, f2W_ref, f2b_ref, f2aW_ref, f2ab_ref,
                f2ns_ref, f2noise_ref, out_ref):
    parts = parts_ref[...]
    agg = parts[0, :_N, :] + parts[1, :_N, :] + edge_b_ref[...]
    out = agg + jnp.dot(agg, c1W_ref[...],
                        preferred_element_type=jnp.float32) + c1b_ref[...]
    xh = jnp.dot(x_ref[...], nmW_ref[...],
                 preferred_element_type=jnp.float32) + nmb_ref[...]
    xn = _style(xh, w_ref[...], nmaW_ref[...], nmab_ref[...],
                nmns_ref[...], nmnoise_ref[...])
    out = out + xn + jnp.dot(xn, c2W_ref[...],
                             preferred_element_type=jnp.float32) + c2b_ref[...]
    out = jnp.maximum(out, 0.0)
    h1 = jnp.dot(out, f1W_ref[...],
                 preferred_element_type=jnp.float32) + f1b_ref[...]
    out = _style(h1, w_ref[...], f1aW_ref[...], f1ab_ref[...],
                 f1ns_ref[...], f1noise_ref[...])
    h2 = jnp.dot(out, f2W_ref[...],
                 preferred_element_type=jnp.float32) + f2b_ref[...]
    out = _style(h2, w_ref[...], f2aW_ref[...], f2ab_ref[...],
                 f2ns_ref[...], f2noise_ref[...])
    out_ref[...] = out


_dense = pl.pallas_call(
    _dense_body,
    out_shape=jax.ShapeDtypeStruct((_N, _H), jnp.float32),
)


def kernel(x, edge_index, w, edge_W, edge_b, cat1_W, cat1_b, cat2_W, cat2_b,
           nm_W, nm_b, nm_aW, nm_ab, nm_ns, nm_noise, f1_W, f1_b, f1_aW,
           f1_ab, f1_ns, f1_noise, f2_W, f2_b, f2_aW, f2_ab, f2_ns, f2_noise):
    src = edge_index[0]
    dst = edge_index[1]
    pad = _EP - _E
    srcp = jnp.concatenate(
        [src, jnp.zeros((pad,), jnp.int32)]).reshape(_NW, 2, _NH, _CH)
    dstp = jnp.concatenate(
        [dst, jnp.full((pad,), _N, jnp.int32)]).reshape(_NW, 2, _NH, _CH)
    extra = jnp.zeros((_NW, 2, 1, _CH), jnp.int32)
    srcp = jnp.concatenate([srcp, extra], axis=2)
    dstp = jnp.concatenate([dstp, extra], axis=2)
    zeros = jnp.zeros((_RPT, _H), jnp.float32)
    parts = _sc_agg(srcp, dstp, edge_W, zeros)
    return _dense(parts, x, w,
                  edge_b.reshape(1, _H), cat1_W, cat1_b.reshape(1, _H),
                  cat2_W, cat2_b.reshape(1, _H), nm_W, nm_b.reshape(1, _H),
                  nm_aW, nm_ab.reshape(1, 2 * _H), nm_ns.reshape(1, 1),
                  nm_noise, f1_W, f1_b.reshape(1, _H), f1_aW,
                  f1_ab.reshape(1, 2 * _H), f1_ns.reshape(1, 1), f1_noise,
                  f2_W, f2_b.reshape(1, _H), f2_aW, f2_ab.reshape(1, 2 * _H),
                  f2_ns.reshape(1, 1), f2_noise)


# speedup vs baseline: 2.8914x; 2.8914x over previous
import functools

import jax
import jax.numpy as jnp
from jax import lax
from jax.experimental import pallas as pl
from jax.experimental.pallas import tpu as pltpu
from jax.experimental.pallas import tpu_sc as plsc

_N = 10000
_E = 320000
_H = 128
_EPS = 1e-5

_NC = 2
_NS = 16
_NW = _NC * _NS
_CH = 128
_Q0 = 101
_Q1 = 56
_NCHUNKS_TOT = _NS * (_Q0 + _Q1)
_RPT = 632
_AGG_ROWS = _NS * _RPT


def _sc_agg_body(src_hbm, dst_hbm, table_hbm, zeros_hbm, out_hbm,
                 src_v, dst_v, rows_v, agg_sh, sem):
    c = lax.axis_index("c")
    s = lax.axis_index("s")
    wid = c * _NS + s
    pltpu.sync_copy(src_hbm.at[wid], src_v)
    pltpu.sync_copy(dst_hbm.at[wid], dst_v)
    base = s * _RPT
    pltpu.sync_copy(zeros_hbm, agg_sh.at[pl.ds(base, _RPT)])
    plsc.subcore_barrier()

    def chunk(j, carry):
        pltpu.async_copy(table_hbm.at[src_v.at[j]], rows_v, sem).wait()
        pltpu.sync_copy(rows_v, agg_sh.at[dst_v.at[j]], add=True)
        return carry

    nch = _Q0 - c * (_Q0 - _Q1)
    lax.fori_loop(0, nch, chunk, 0)
    plsc.subcore_barrier()
    pltpu.sync_copy(agg_sh.at[pl.ds(base, _RPT)],
                    out_hbm.at[c, pl.ds(base, _RPT)])


_sc_agg = pl.kernel(
    _sc_agg_body,
    out_type=jax.ShapeDtypeStruct((_NC, _AGG_ROWS, _H), jnp.float32),
    mesh=plsc.VectorSubcoreMesh(core_axis_name="c", subcore_axis_name="s"),
    scratch_types=[
        pltpu.VMEM((_Q0, _CH), jnp.int32),
        pltpu.VMEM((_Q0, _CH), jnp.int32),
        pltpu.VMEM((_CH, _H), jnp.float32),
        pltpu.VMEM_SHARED((_AGG_ROWS, _H), jnp.float32),
        pltpu.SemaphoreType.DMA,
    ],
)


def _style(h, wv, aW, ab, ns, noise):
    h = h + noise * ns
    st = jnp.dot(wv, aW, preferred_element_type=jnp.float32) + ab
    gamma = st[:, :_H]
    beta = st[:, _H:]
    mu = jnp.mean(h, axis=0, keepdims=True)
    d = h - mu
    var = jnp.mean(d * d, axis=0, keepdims=True)
    hn = d * lax.rsqrt(var + _EPS)
    o = gamma * hn + beta
    return jnp.where(o >= 0, o, 0.01 * o)


def _dense_body(parts_ref, x_ref, w_ref, edge_b_ref, c1W_ref, c1b_ref,
                c2W_ref, c2b_ref, nmW_ref, nmb_ref, nmaW_ref, nmab_ref,
                nmns_ref, nmnoise_ref, f1W_ref, f1b_ref, f1aW_ref, f1ab_ref,
                f1ns_ref, f1noise_ref, f2W_ref, f2b_ref, f2aW_ref, f2ab_ref,
                f2ns_ref, f2noise_ref, out_ref):
    parts = parts_ref[...]
    agg = parts[0, :_N, :] + parts[1, :_N, :] + edge_b_ref[...]
    out = agg + jnp.dot(agg, c1W_ref[...],
                        preferred_element_type=jnp.float32) + c1b_ref[...]
    xh = jnp.dot(x_ref[...], nmW_ref[...],
                 preferred_element_type=jnp.float32) + nmb_ref[...]
    xn = _style(xh, w_ref[...], nmaW_ref[...], nmab_ref[...],
                nmns_ref[...], nmnoise_ref[...])
    out = out + xn + jnp.dot(xn, c2W_ref[...],
                             preferred_element_type=jnp.float32) + c2b_ref[...]
    out = jnp.maximum(out, 0.0)
    h1 = jnp.dot(out, f1W_ref[...],
                 preferred_element_type=jnp.float32) + f1b_ref[...]
    out = _style(h1, w_ref[...], f1aW_ref[...], f1ab_ref[...],
                 f1ns_ref[...], f1noise_ref[...])
    h2 = jnp.dot(out, f2W_ref[...],
                 preferred_element_type=jnp.float32) + f2b_ref[...]
    out = _style(h2, w_ref[...], f2aW_ref[...], f2ab_ref[...],
                 f2ns_ref[...], f2noise_ref[...])
    out_ref[...] = out


_dense = pl.pallas_call(
    _dense_body,
    out_shape=jax.ShapeDtypeStruct((_N, _H), jnp.float32),
)


def kernel(x, edge_index, w, edge_W, edge_b, cat1_W, cat1_b, cat2_W, cat2_b,
           nm_W, nm_b, nm_aW, nm_ab, nm_ns, nm_noise, f1_W, f1_b, f1_aW,
           f1_ab, f1_ns, f1_noise, f2_W, f2_b, f2_aW, f2_ab, f2_ns, f2_noise):
    src = edge_index[0]
    dst = edge_index[1]
    pad = _NCHUNKS_TOT * _CH - _E

    def _layout(a, fill):
        a = jnp.concatenate([a, jnp.full((pad,), fill, jnp.int32)])
        a0 = a[:_NS * _Q0 * _CH].reshape(_NS, _Q0, _CH)
        a1 = a[_NS * _Q0 * _CH:].reshape(_NS, _Q1, _CH)
        a1 = jnp.pad(a1, ((0, 0), (0, _Q0 - _Q1), (0, 0)),
                     constant_values=fill)
        return jnp.concatenate([a0, a1], axis=0)

    srcp = _layout(src, 0)
    dstp = _layout(dst, _N)
    zeros = jnp.zeros((_RPT, _H), jnp.float32)
    parts = _sc_agg(srcp, dstp, edge_W, zeros)
    return _dense(parts, x, w,
                  edge_b.reshape(1, _H), cat1_W, cat1_b.reshape(1, _H),
                  cat2_W, cat2_b.reshape(1, _H), nm_W, nm_b.reshape(1, _H),
                  nm_aW, nm_ab.reshape(1, 2 * _H), nm_ns.reshape(1, 1),
                  nm_noise, f1_W, f1_b.reshape(1, _H), f1_aW,
                  f1_ab.reshape(1, 2 * _H), f1_ns.reshape(1, 1), f1_noise,
                  f2_W, f2_b.reshape(1, _H), f2_aW, f2_ab.reshape(1, 2 * _H),
                  f2_ns.reshape(1, 1), f2_noise)
